# baseline (speedup 1.0000x reference)
import functools

import jax
import jax.numpy as jnp
from jax import lax
from jax.experimental import pallas as pl
from jax.experimental.pallas import tpu as pltpu
from jax.experimental.pallas import tpu_sc as plsc

_NC = 2
_NS = 16
_NW = _NC * _NS
_SEG = 31744
_CH = 256
_TAIL0 = _NW - 1
_LCAP = 16400
_STG = 64


def _iota16():
    return lax.broadcasted_iota(jnp.int32, (16,), 0)


def _emb_lookup(tableC, labels, tail2, V, B):
    tail_r0 = _TAIL0 * _SEG + 62 * _CH
    n_tail = V - tail_r0
    mesh = plsc.VectorSubcoreMesh(core_axis_name="c", subcore_axis_name="s")

    @functools.partial(
        pl.kernel,
        out_type=jax.ShapeDtypeStruct((B + 32, 128), jnp.float32),
        mesh=mesh,
        compiler_params=pltpu.CompilerParams(
            use_tc_tiling_on_sc=True, needs_layout_passes=False
        ),
        scratch_types=[
            pltpu.VMEM((2048,), jnp.int32),
            pltpu.VMEM((_LCAP,), jnp.int32),
            pltpu.VMEM((_LCAP,), jnp.int32),
            pltpu.VMEM((_LCAP,), jnp.int32),
            pltpu.VMEM((2, 8, 8, _CH), jnp.float32),
            pltpu.VMEM((128, 64), jnp.float32),
            pltpu.VMEM((_STG, 128), jnp.float32),
            pltpu.VMEM((_STG,), jnp.int32),
            pltpu.SemaphoreType.DMA,
            pltpu.SemaphoreType.DMA,
        ],
    )
    def body(tab, idx, tl, out, lab_blk, mpack, spack, cpack, ring, tailv,
             stag, bstage, sem_c, sem_s):
        wid = lax.axis_index("s") * _NC + lax.axis_index("c")
        lo = wid * _SEG
        n_ch = jnp.where(wid == _TAIL0, 62, 124)
        hi = lo + n_ch * _CH
        iota = _iota16()
        is0 = wid == 0
        d0 = [(iota + j * 16) >> 3 for j in range(4)]
        d1 = [(iota + j * 16) & 7 for j in range(4)]

        def scan_block(blk, off):
            pltpu.sync_copy(idx.at[pl.ds(blk * 2048, 2048)], lab_blk)

            def scan_i(i, off):
                lab = lab_blk[pl.ds(i * 16, 16)]
                tail_m = is0 & (lab >= tail_r0)
                m = ((lab >= lo) & (lab < hi)) | tail_m
                cnt = jnp.sum(jnp.where(m, 1, 0).astype(jnp.int32))
                loff = jnp.where(tail_m, lab - tail_r0 + _SEG, lab - lo)
                bv = iota + (blk * 2048 + i * 16)
                packed = (loff << 14) | bv
                plsc.store_compressed(mpack.at[pl.ds(off, 16)], packed, mask=m)
                return off + cnt

            return lax.fori_loop(0, 128, scan_i, off)

        n_m = lax.fori_loop(0, 8, scan_block, 0)

        for k in range(_STG // 16):
            bstage[pl.ds(k * 16, 16)] = iota + B

        def fire():
            pltpu.async_copy(stag, out.at[bstage], sem_s).wait()

        def compress(src_ref, n_src, dst_ref, shift, want):
            def comp_j(j, cc):
                pk = src_ref[pl.ds(j * 16, 16)]
                valid = (iota + j * 16) < n_src
                m2 = ((pk >> shift) == want) & valid
                cnt = jnp.sum(jnp.where(m2, 1, 0).astype(jnp.int32))
                plsc.store_compressed(dst_ref.at[pl.ds(cc, 16)], pk, mask=m2)
                return cc + cnt

            return lax.fori_loop(0, (n_src + 15) // 16, comp_j, 0)

        def extract(gather_fn, base_off, width, n_c, staged):
            def grp(g, st):
                pk = cpack[pl.ds(g * 16, 16)]
                valid = (iota + g * 16) < n_c
                bvs = jnp.where(valid, pk & 16383, iota + B)
                rloc = (pk >> 14) - base_off
                rloc = jnp.minimum(jnp.maximum(rloc, 0), width - 1)
                row0 = st % _STG
                bstage[pl.ds(row0, 16)] = bvs
                for i in range(16):
                    rs = jnp.broadcast_to(rloc[i], (16,))
                    rowv = jnp.broadcast_to(row0 + i, (16,))
                    for j in range(4):
                        vals = gather_fn(rs, d0[j], d1[j], j)
                        plsc.store_scatter(stag, [rowv, iota + j * 16], vals)
                st = st + 16

                @pl.when(st % _STG == 0)
                def _():
                    fire()

                return st

            return lax.fori_loop(0, (n_c + 15) // 16, grp, staged)

        pltpu.async_copy(
            tab.at[:, :, pl.ds(lo, _CH)], ring.at[0], sem_c
        )

        def chunk_body(c, carry):
            staged, n_s = carry
            slot = lax.rem(c, 2)
            r0 = lo + c * _CH

            @pl.when(c + 1 < n_ch)
            def _():
                pltpu.async_copy(
                    tab.at[:, :, pl.ds(r0 + _CH, _CH)],
                    ring.at[lax.rem(c + 1, 2)],
                    sem_c,
                )

            n_s = lax.cond(
                lax.rem(c, 16) == 0,
                lambda: compress(mpack, n_m, spack, 26, c // 16),
                lambda: n_s,
            )
            n_c = compress(spack, n_s, cpack, 22, c)

            pltpu.make_async_copy(
                tab.at[:, :, pl.ds(r0, _CH)], ring.at[slot], sem_c
            ).wait()

            def gather_ring(rs, dj0, dj1, j):
                return plsc.load_gather(ring.at[slot], [dj0, dj1, rs])

            staged = extract(gather_ring, c * _CH, _CH, n_c, staged)
            return staged, n_s

        staged, _ = lax.fori_loop(0, n_ch, chunk_body, (0, 0))

        @pl.when(is0)
        def _():
            pltpu.sync_copy(tl, tailv)
            n_c = compress(mpack, n_m, cpack, 22, _SEG // _CH)

            def gather_tail(rs, dj0, dj1, j):
                return plsc.load_gather(tailv, [rs, iota + j * 16])

            extract(gather_tail, _SEG, n_tail, n_c, staged)

        fire()

    return body(tableC, labels, tail2)


def kernel(labels, embedding_table):
    B, = labels.shape
    V, D = embedding_table.shape
    tableC = embedding_table.T.reshape(8, D // 8, V)
    tail_r0 = _TAIL0 * _SEG + 62 * _CH
    tail2 = jnp.pad(embedding_table[tail_r0:], ((0, 128 - (V - tail_r0)), (0, 0)))
    out128 = _emb_lookup(tableC, labels.astype(jnp.int32), tail2, V, B)
    return out128[:B, :D]


# speedup vs baseline: 1.3819x; 1.3819x over previous
import functools

import jax
import jax.numpy as jnp
from jax import lax
from jax.experimental import pallas as pl
from jax.experimental.pallas import tpu as pltpu
from jax.experimental.pallas import tpu_sc as plsc

_NC = 2
_NS = 16
_NW = _NC * _NS
_SEG = 31744
_CH = 512
_TAIL0 = _NW - 1
_LCAP = 16400
_STG = 48


def _iota16():
    return lax.broadcasted_iota(jnp.int32, (16,), 0)


def _emb_lookup(tableC, labels, tail2, V, B):
    tail_r0 = _TAIL0 * _SEG + 31 * _CH
    n_tail = V - tail_r0
    mesh = plsc.VectorSubcoreMesh(core_axis_name="c", subcore_axis_name="s")

    @functools.partial(
        pl.kernel,
        out_type=jax.ShapeDtypeStruct((B + 32, 128), jnp.float32),
        mesh=mesh,
        compiler_params=pltpu.CompilerParams(
            use_tc_tiling_on_sc=True, needs_layout_passes=False
        ),
        scratch_types=[
            pltpu.VMEM((1024,), jnp.int32),
            pltpu.VMEM((_LCAP,), jnp.int32),
            pltpu.VMEM((_LCAP,), jnp.int32),
            pltpu.VMEM((_LCAP,), jnp.int32),
            pltpu.VMEM((2, 8, 8, _CH), jnp.float32),
            pltpu.VMEM((_STG, 128), jnp.float32),
            pltpu.VMEM((_STG,), jnp.int32),
            pltpu.SemaphoreType.DMA,
            pltpu.SemaphoreType.DMA,
        ],
    )
    def body(tab, idx, tl, out, lab_blk, mpack, spack, cpack, ring,
             stag, bstage, sem_c, sem_s):
        wid = lax.axis_index("s") * _NC + lax.axis_index("c")
        lo = wid * _SEG
        n_ch = jnp.where(wid == _TAIL0, 31, 62)
        hi = lo + n_ch * _CH
        iota = _iota16()
        is0 = wid == 0
        d0 = [(iota + j * 16) >> 3 for j in range(4)]
        d1 = [(iota + j * 16) & 7 for j in range(4)]

        def scan_block(blk, off):
            pltpu.sync_copy(idx.at[pl.ds(blk * 1024, 1024)], lab_blk)

            def scan_i(i, off):
                lab = lab_blk[pl.ds(i * 16, 16)]
                tail_m = is0 & (lab >= tail_r0)
                m = ((lab >= lo) & (lab < hi)) | tail_m
                cnt = jnp.sum(jnp.where(m, 1, 0).astype(jnp.int32))
                loff = jnp.where(tail_m, lab - tail_r0 + _SEG, lab - lo)
                bv = iota + (blk * 1024 + i * 16)
                packed = (loff << 14) | bv
                plsc.store_compressed(mpack.at[pl.ds(off, 16)], packed, mask=m)
                return off + cnt

            return lax.fori_loop(0, 64, scan_i, off)

        n_m = lax.fori_loop(0, 16, scan_block, 0)

        for k in range(_STG // 16):
            bstage[pl.ds(k * 16, 16)] = iota + B

        def fire():
            pltpu.async_copy(stag, out.at[bstage], sem_s).wait()

        def compress(src_ref, n_src, dst_ref, shift, want):
            def comp_j(j, cc):
                pk = src_ref[pl.ds(j * 16, 16)]
                valid = (iota + j * 16) < n_src
                m2 = ((pk >> shift) == want) & valid
                cnt = jnp.sum(jnp.where(m2, 1, 0).astype(jnp.int32))
                plsc.store_compressed(dst_ref.at[pl.ds(cc, 16)], pk, mask=m2)
                return cc + cnt

            return lax.fori_loop(0, (n_src + 15) // 16, comp_j, 0)

        def extract(gather_fn, base_off, width, n_c, staged):
            def grp(g, st):
                pk = cpack[pl.ds(g * 16, 16)]
                valid = (iota + g * 16) < n_c
                bvs = jnp.where(valid, pk & 16383, iota + B)
                rloc = (pk >> 14) - base_off
                rloc = jnp.minimum(jnp.maximum(rloc, 0), width - 1)
                row0 = st % _STG
                bstage[pl.ds(row0, 16)] = bvs
                for i in range(16):
                    rs = jnp.broadcast_to(rloc[i], (16,))
                    rowv = jnp.broadcast_to(row0 + i, (16,))
                    for j in range(4):
                        vals = gather_fn(rs, d0[j], d1[j], j)
                        plsc.store_scatter(stag, [rowv, iota + j * 16], vals)
                st = st + 16

                @pl.when(st % _STG == 0)
                def _():
                    fire()

                return st

            return lax.fori_loop(0, (n_c + 15) // 16, grp, staged)

        pltpu.async_copy(
            tab.at[:, :, pl.ds(lo, _CH)], ring.at[0], sem_c
        )

        def chunk_body(c, carry):
            staged, n_s = carry
            slot = lax.rem(c, 2)
            r0 = lo + c * _CH

            @pl.when(c + 1 < n_ch)
            def _():
                pltpu.async_copy(
                    tab.at[:, :, pl.ds(r0 + _CH, _CH)],
                    ring.at[lax.rem(c + 1, 2)],
                    sem_c,
                )

            n_s = lax.cond(
                lax.rem(c, 8) == 0,
                lambda: compress(mpack, n_m, spack, 26, c // 8),
                lambda: n_s,
            )
            n_c = compress(spack, n_s, cpack, 23, c)

            pltpu.make_async_copy(
                tab.at[:, :, pl.ds(r0, _CH)], ring.at[slot], sem_c
            ).wait()

            def gather_ring(rs, dj0, dj1, j):
                return plsc.load_gather(ring.at[slot], [dj0, dj1, rs])

            staged = extract(gather_ring, c * _CH, _CH, n_c, staged)
            return staged, n_s

        staged, _ = lax.fori_loop(0, n_ch, chunk_body, (0, 0))

        @pl.when(is0)
        def _():
            pltpu.sync_copy(tl, ring.at[0].at[:, :, pl.ds(0, 128)])
            n_c = compress(mpack, n_m, cpack, 23, _SEG // _CH)

            def gather_tail(rs, dj0, dj1, j):
                return plsc.load_gather(ring.at[0], [dj0, dj1, rs])

            extract(gather_tail, _SEG, n_tail, n_c, staged)

        fire()

    return body(tableC, labels, tail2)


def kernel(labels, embedding_table):
    B, = labels.shape
    V, D = embedding_table.shape
    tableC = embedding_table.T.reshape(8, D // 8, V)
    tail_r0 = _TAIL0 * _SEG + 31 * _CH
    tail2 = jnp.pad(embedding_table[tail_r0:], ((0, 128 - (V - tail_r0)), (0, 0)))
    tail2 = tail2.T.reshape(8, D // 8, 128)
    out128 = _emb_lookup(tableC, labels.astype(jnp.int32), tail2, V, B)
    return out128[:B, :D]


# speedup vs baseline: 1.4171x; 1.0255x over previous
import functools

import jax
import jax.numpy as jnp
from jax import lax
from jax.experimental import pallas as pl
from jax.experimental.pallas import tpu as pltpu
from jax.experimental.pallas import tpu_sc as plsc

_NC = 2
_NS = 16
_NW = _NC * _NS
_SEG = 31744
_CH = 512
_TAIL0 = _NW - 1
_LCAP = 16400
_STG = 48


def _iota16():
    return lax.broadcasted_iota(jnp.int32, (16,), 0)


def _emb_lookup(tableC, labels, tail2, V, B):
    tail_r0 = _TAIL0 * _SEG + 31 * _CH
    n_tail = V - tail_r0
    mesh = plsc.VectorSubcoreMesh(core_axis_name="c", subcore_axis_name="s")

    @functools.partial(
        pl.kernel,
        out_type=jax.ShapeDtypeStruct((B + 32, 128), jnp.float32),
        mesh=mesh,
        compiler_params=pltpu.CompilerParams(
            use_tc_tiling_on_sc=True, needs_layout_passes=False
        ),
        scratch_types=[
            pltpu.VMEM((4096,), jnp.int32),
            pltpu.VMEM((_LCAP,), jnp.int32),
            pltpu.VMEM((_LCAP,), jnp.int32),
            pltpu.VMEM((_LCAP,), jnp.int32),
            pltpu.VMEM((2, 8, 8, _CH), jnp.float32),
            pltpu.VMEM((_STG, 128), jnp.float32),
            pltpu.VMEM((_STG,), jnp.int32),
            pltpu.SemaphoreType.DMA,
            pltpu.SemaphoreType.DMA,
        ],
    )
    def body(tab, idx, tl, out, lab_blk, mpack, spack, cpack, ring,
             stag, bstage, sem_c, sem_s):
        wid = lax.axis_index("s") * _NC + lax.axis_index("c")
        lo = wid * _SEG

        def load_chunk(r0, slot):
            for ct in range(8):
                pltpu.async_copy(
                    tab.at[ct].at[:, pl.ds(r0, _CH)],
                    ring.at[slot].at[ct],
                    sem_c,
                )

        load_chunk(lo, 0)
        n_ch = jnp.where(wid == _TAIL0, 31, 62)
        hi = lo + n_ch * _CH
        iota = _iota16()
        is0 = wid == 0
        d0 = [(iota + j * 16) >> 3 for j in range(4)]
        d1 = [(iota + j * 16) & 7 for j in range(4)]

        def scan_block(blk, off):
            pltpu.sync_copy(idx.at[pl.ds(blk * 4096, 4096)], lab_blk)

            def scan_i(i, off):
                lab = lab_blk[pl.ds(i * 16, 16)]
                tail_m = is0 & (lab >= tail_r0)
                m = ((lab >= lo) & (lab < hi)) | tail_m
                cnt = jnp.sum(jnp.where(m, 1, 0).astype(jnp.int32))
                loff = jnp.where(tail_m, lab - tail_r0 + _SEG, lab - lo)
                bv = iota + (blk * 4096 + i * 16)
                packed = (loff << 14) | bv
                plsc.store_compressed(mpack.at[pl.ds(off, 16)], packed, mask=m)
                return off + cnt

            return lax.fori_loop(0, 256, scan_i, off)

        n_m = lax.fori_loop(0, 4, scan_block, 0)

        for k in range(_STG // 16):
            bstage[pl.ds(k * 16, 16)] = iota + B

        def fire():
            pltpu.async_copy(stag, out.at[bstage], sem_s).wait()

        def compress(src_ref, n_src, dst_ref, shift, want):
            def comp_j(j, cc):
                pk = src_ref[pl.ds(j * 16, 16)]
                valid = (iota + j * 16) < n_src
                m2 = ((pk >> shift) == want) & valid
                cnt = jnp.sum(jnp.where(m2, 1, 0).astype(jnp.int32))
                plsc.store_compressed(dst_ref.at[pl.ds(cc, 16)], pk, mask=m2)
                return cc + cnt

            return lax.fori_loop(0, (n_src + 15) // 16, comp_j, 0)

        def extract(gather_fn, base_off, width, n_c, staged):
            def grp(g, st):
                pk = cpack[pl.ds(g * 16, 16)]
                valid = (iota + g * 16) < n_c
                bvs = jnp.where(valid, pk & 16383, iota + B)
                rloc = (pk >> 14) - base_off
                rloc = jnp.minimum(jnp.maximum(rloc, 0), width - 1)
                row0 = st % _STG
                bstage[pl.ds(row0, 16)] = bvs
                for i in range(16):
                    rs = jnp.broadcast_to(rloc[i], (16,))
                    rowv = jnp.broadcast_to(row0 + i, (16,))
                    for j in range(4):
                        vals = gather_fn(rs, d0[j], d1[j], j)
                        plsc.store_scatter(stag, [rowv, iota + j * 16], vals)
                st = st + 16

                @pl.when(st % _STG == 0)
                def _():
                    fire()

                return st

            return lax.fori_loop(0, (n_c + 15) // 16, grp, staged)


        def chunk_body(c, carry):
            staged, n_s = carry
            slot = lax.rem(c, 2)
            r0 = lo + c * _CH

            @pl.when(c + 1 < n_ch)
            def _():
                load_chunk(r0 + _CH, lax.rem(c + 1, 2))

            n_s = lax.cond(
                lax.rem(c, 8) == 0,
                lambda: compress(mpack, n_m, spack, 26, c // 8),
                lambda: n_s,
            )
            n_c = compress(spack, n_s, cpack, 23, c)

            pltpu.make_async_copy(
                tab.at[:, :, pl.ds(r0, _CH)], ring.at[slot], sem_c
            ).wait()

            def gather_ring(rs, dj0, dj1, j):
                return plsc.load_gather(ring.at[slot], [dj0, dj1, rs])

            staged = extract(gather_ring, c * _CH, _CH, n_c, staged)
            return staged, n_s

        staged, _ = lax.fori_loop(0, n_ch, chunk_body, (0, 0))

        @pl.when(is0)
        def _():
            pltpu.sync_copy(tl, ring.at[0].at[:, :, pl.ds(0, 128)])
            n_c = compress(mpack, n_m, cpack, 23, _SEG // _CH)

            def gather_tail(rs, dj0, dj1, j):
                return plsc.load_gather(ring.at[0], [dj0, dj1, rs])

            extract(gather_tail, _SEG, n_tail, n_c, staged)

        fire()

    return body(tableC, labels, tail2)


def kernel(labels, embedding_table):
    B, = labels.shape
    V, D = embedding_table.shape
    tableC = embedding_table.T.reshape(8, D // 8, V)
    tail_r0 = _TAIL0 * _SEG + 31 * _CH
    tail2 = jnp.pad(embedding_table[tail_r0:], ((0, 128 - (V - tail_r0)), (0, 0)))
    tail2 = tail2.T.reshape(8, D // 8, 128)
    out128 = _emb_lookup(tableC, labels.astype(jnp.int32), tail2, V, B)
    return out128[:B, :D]


# speedup vs baseline: 1.4198x; 1.0019x over previous
import functools

import jax
import jax.numpy as jnp
from jax import lax
from jax.experimental import pallas as pl
from jax.experimental.pallas import tpu as pltpu
from jax.experimental.pallas import tpu_sc as plsc

_NC = 2
_NS = 16
_NW = _NC * _NS
_SEG = 31744
_CH = 512
_TAIL0 = _NW - 1
_LCAP = 16400
_STG = 48


def _iota16():
    return lax.broadcasted_iota(jnp.int32, (16,), 0)


def _emb_lookup(tableC, labels, tail2, V, B):
    tail_r0 = _TAIL0 * _SEG + 31 * _CH
    n_tail = V - tail_r0
    mesh = plsc.VectorSubcoreMesh(core_axis_name="c", subcore_axis_name="s")

    @functools.partial(
        pl.kernel,
        out_type=jax.ShapeDtypeStruct((B + 32, 128), jnp.float32),
        mesh=mesh,
        compiler_params=pltpu.CompilerParams(
            use_tc_tiling_on_sc=True, needs_layout_passes=False
        ),
        scratch_types=[
            pltpu.VMEM((4096,), jnp.int32),
            pltpu.VMEM((_LCAP,), jnp.int32),
            pltpu.VMEM((_LCAP,), jnp.int32),
            pltpu.VMEM((_LCAP,), jnp.int32),
            pltpu.VMEM((2, 8, 8, _CH), jnp.float32),
            pltpu.VMEM((_STG, 128), jnp.float32),
            pltpu.VMEM((_STG,), jnp.int32),
            pltpu.SemaphoreType.DMA,
            pltpu.SemaphoreType.DMA,
        ],
    )
    def body(tab, idx, tl, out, lab_blk, mpack, spack, cpack, ring,
             stag, bstage, sem_c, sem_s):
        wid = lax.axis_index("s") * _NC + lax.axis_index("c")
        lo = wid * _SEG

        def load_chunk(r0, slot):
            for ct in range(8):
                pltpu.async_copy(
                    tab.at[ct].at[:, pl.ds(r0, _CH)],
                    ring.at[slot].at[ct],
                    sem_c,
                )

        load_chunk(lo, 0)
        n_ch = jnp.where(wid == _TAIL0, 31, 62)
        hi = lo + n_ch * _CH
        iota = _iota16()
        is0 = wid == 0

        def scan_block(blk, off):
            pltpu.sync_copy(idx.at[pl.ds(blk * 4096, 4096)], lab_blk)

            def scan_i(i, off):
                lab = lab_blk[pl.ds(i * 16, 16)]
                tail_m = is0 & (lab >= tail_r0)
                m = ((lab >= lo) & (lab < hi)) | tail_m
                cnt = jnp.sum(jnp.where(m, 1, 0).astype(jnp.int32))
                loff = jnp.where(tail_m, lab - tail_r0 + _SEG, lab - lo)
                bv = iota + (blk * 4096 + i * 16)
                packed = (loff << 14) | bv
                plsc.store_compressed(mpack.at[pl.ds(off, 16)], packed, mask=m)
                return off + cnt

            return lax.fori_loop(0, 256, scan_i, off)

        n_m = lax.fori_loop(0, 4, scan_block, 0)

        for k in range(_STG // 16):
            bstage[pl.ds(k * 16, 16)] = iota + B

        def fire():
            pltpu.async_copy(stag, out.at[bstage], sem_s).wait()

        def compress(src_ref, n_src, dst_ref, shift, want):
            def comp_j(j, cc):
                pk = src_ref[pl.ds(j * 16, 16)]
                valid = (iota + j * 16) < n_src
                m2 = ((pk >> shift) == want) & valid
                cnt = jnp.sum(jnp.where(m2, 1, 0).astype(jnp.int32))
                plsc.store_compressed(dst_ref.at[pl.ds(cc, 16)], pk, mask=m2)
                return cc + cnt

            return lax.fori_loop(0, (n_src + 15) // 16, comp_j, 0)

        def extract(gather_fn, base_off, width, n_c, staged):
            def grp(g, st):
                pk = cpack[pl.ds(g * 16, 16)]
                valid = (iota + g * 16) < n_c
                bvs = jnp.where(valid, pk & 16383, iota + B)
                rloc = (pk >> 14) - base_off
                rloc = jnp.minimum(jnp.maximum(rloc, 0), width - 1)
                row0 = st % _STG
                bstage[pl.ds(row0, 16)] = bvs
                rowi = iota + row0
                for ct in range(8):
                    i0 = jnp.full((16,), ct, jnp.int32)
                    for s in range(8):
                        i1 = jnp.full((16,), s, jnp.int32)
                        vals = gather_fn(i0, i1, rloc)
                        plsc.store_scatter(stag, [rowi, i0 * 8 + i1], vals)
                st = st + 16

                @pl.when(st % _STG == 0)
                def _():
                    fire()

                return st

            return lax.fori_loop(0, (n_c + 15) // 16, grp, staged)


        def chunk_body(c, carry):
            staged, n_s = carry
            slot = lax.rem(c, 2)
            r0 = lo + c * _CH

            @pl.when(c + 1 < n_ch)
            def _():
                load_chunk(r0 + _CH, lax.rem(c + 1, 2))

            n_s = lax.cond(
                lax.rem(c, 8) == 0,
                lambda: compress(mpack, n_m, spack, 26, c // 8),
                lambda: n_s,
            )
            n_c = compress(spack, n_s, cpack, 23, c)

            pltpu.make_async_copy(
                tab.at[:, :, pl.ds(r0, _CH)], ring.at[slot], sem_c
            ).wait()

            def gather_ring(i0, i1, rloc):
                return plsc.load_gather(ring.at[slot], [i0, i1, rloc])

            staged = extract(gather_ring, c * _CH, _CH, n_c, staged)
            return staged, n_s

        staged, _ = lax.fori_loop(0, n_ch, chunk_body, (0, 0))

        @pl.when(is0)
        def _():
            pltpu.sync_copy(tl, ring.at[0].at[:, :, pl.ds(0, 128)])
            n_c = compress(mpack, n_m, cpack, 23, _SEG // _CH)

            def gather_tail(i0, i1, rloc):
                return plsc.load_gather(ring.at[0], [i0, i1, rloc])

            extract(gather_tail, _SEG, n_tail, n_c, staged)

        fire()

    return body(tableC, labels, tail2)


def kernel(labels, embedding_table):
    B, = labels.shape
    V, D = embedding_table.shape
    tableC = embedding_table.T.reshape(8, D // 8, V)
    tail_r0 = _TAIL0 * _SEG + 31 * _CH
    tail2 = jnp.pad(embedding_table[tail_r0:], ((0, 128 - (V - tail_r0)), (0, 0)))
    tail2 = tail2.T.reshape(8, D // 8, 128)
    out128 = _emb_lookup(tableC, labels.astype(jnp.int32), tail2, V, B)
    return out128[:B, :D]


# speedup vs baseline: 1.4378x; 1.0127x over previous
import functools

import jax
import jax.numpy as jnp
from jax import lax
from jax.experimental import pallas as pl
from jax.experimental.pallas import tpu as pltpu
from jax.experimental.pallas import tpu_sc as plsc

_NC = 2
_NS = 16
_NW = _NC * _NS
_SEG = 31744
_CH = 512
_TAIL0 = _NW - 1
_LCAP = 16400
_STG = 48


def _iota16():
    return lax.broadcasted_iota(jnp.int32, (16,), 0)


def _emb_lookup(tableC, labels, tail2, V, B):
    tail_r0 = _TAIL0 * _SEG + 31 * _CH
    n_tail = V - tail_r0
    mesh = plsc.VectorSubcoreMesh(core_axis_name="c", subcore_axis_name="s")

    @functools.partial(
        pl.kernel,
        out_type=jax.ShapeDtypeStruct((B + 32, 128), jnp.float32),
        mesh=mesh,
        compiler_params=pltpu.CompilerParams(
            use_tc_tiling_on_sc=True, needs_layout_passes=False
        ),
        scratch_types=[
            pltpu.VMEM((2048,), jnp.int32),
            pltpu.VMEM((_LCAP,), jnp.int32),
            pltpu.VMEM((_LCAP,), jnp.int32),
            pltpu.VMEM((_LCAP,), jnp.int32),
            pltpu.VMEM((2, 8, 8, _CH), jnp.float32),
            pltpu.VMEM((2, _STG, 128), jnp.float32),
            pltpu.VMEM((2, _STG), jnp.int32),
            pltpu.SemaphoreType.DMA,
            pltpu.SemaphoreType.DMA,
        ],
    )
    def body(tab, idx, tl, out, lab_blk, mpack, spack, cpack, ring,
             stag, bstage, sem_c, sem_s):
        wid = lax.axis_index("s") * _NC + lax.axis_index("c")
        lo = wid * _SEG

        def load_chunk(r0, slot):
            for ct in range(8):
                pltpu.async_copy(
                    tab.at[ct].at[:, pl.ds(r0, _CH)],
                    ring.at[slot].at[ct],
                    sem_c,
                )

        load_chunk(lo, 0)
        n_ch = jnp.where(wid == _TAIL0, 31, 62)
        hi = lo + n_ch * _CH
        iota = _iota16()
        is0 = wid == 0

        def scan_block(blk, off):
            pltpu.sync_copy(idx.at[pl.ds(blk * 2048, 2048)], lab_blk)

            def scan_i(i, off):
                lab = lab_blk[pl.ds(i * 16, 16)]
                tail_m = is0 & (lab >= tail_r0)
                m = ((lab >= lo) & (lab < hi)) | tail_m
                cnt = jnp.sum(jnp.where(m, 1, 0).astype(jnp.int32))
                loff = jnp.where(tail_m, lab - tail_r0 + _SEG, lab - lo)
                bv = iota + (blk * 2048 + i * 16)
                packed = (loff << 14) | bv
                plsc.store_compressed(mpack.at[pl.ds(off, 16)], packed, mask=m)
                return off + cnt

            return lax.fori_loop(0, 128, scan_i, off)

        n_m = lax.fori_loop(0, 8, scan_block, 0)

        for h in range(2):
            for k in range(_STG // 16):
                bstage.at[h][pl.ds(k * 16, 16)] = iota + B

        def fire_wait():
            pltpu.make_async_copy(
                stag.at[0], out.at[bstage.at[0]], sem_s
            ).wait()

        def final_drain(st):
            hf = lax.rem(st // _STG, 2)
            pltpu.async_copy(stag.at[hf], out.at[bstage.at[hf]], sem_s)
            fire_wait()

            @pl.when(st >= _STG)
            def _():
                fire_wait()

        def compress(src_ref, n_src, dst_ref, shift, want):
            def comp_j(j, cc):
                pk = src_ref[pl.ds(j * 16, 16)]
                valid = (iota + j * 16) < n_src
                m2 = ((pk >> shift) == want) & valid
                cnt = jnp.sum(jnp.where(m2, 1, 0).astype(jnp.int32))
                plsc.store_compressed(dst_ref.at[pl.ds(cc, 16)], pk, mask=m2)
                return cc + cnt

            return lax.fori_loop(0, (n_src + 15) // 16, comp_j, 0)

        def extract(gather_fn, base_off, width, n_c, staged):
            def grp(g, st):
                pk = cpack[pl.ds(g * 16, 16)]
                valid = (iota + g * 16) < n_c
                bvs = jnp.where(valid, pk & 16383, iota + B)
                rloc = (pk >> 14) - base_off
                rloc = jnp.minimum(jnp.maximum(rloc, 0), width - 1)
                h = lax.rem(st // _STG, 2)
                row0 = st % _STG
                bstage.at[h][pl.ds(row0, 16)] = bvs
                rowi = iota + row0
                for ct in range(8):
                    i0 = jnp.full((16,), ct, jnp.int32)
                    for s in range(8):
                        i1 = jnp.full((16,), s, jnp.int32)
                        vals = gather_fn(i0, i1, rloc)
                        plsc.store_scatter(stag.at[h], [rowi, i0 * 8 + i1], vals)
                st = st + 16

                @pl.when(st % _STG == 0)
                def _():
                    hf = lax.rem(st // _STG + 1, 2)
                    pltpu.async_copy(
                        stag.at[hf], out.at[bstage.at[hf]], sem_s
                    )

                    @pl.when(st >= 2 * _STG)
                    def _():
                        fire_wait()

                return st

            return lax.fori_loop(0, (n_c + 15) // 16, grp, staged)


        def chunk_body(c, carry):
            staged, n_s = carry
            slot = lax.rem(c, 2)
            r0 = lo + c * _CH

            @pl.when(c + 1 < n_ch)
            def _():
                load_chunk(r0 + _CH, lax.rem(c + 1, 2))

            n_s = lax.cond(
                lax.rem(c, 8) == 0,
                lambda: compress(mpack, n_m, spack, 26, c // 8),
                lambda: n_s,
            )
            n_c = compress(spack, n_s, cpack, 23, c)

            pltpu.make_async_copy(
                tab.at[:, :, pl.ds(r0, _CH)], ring.at[slot], sem_c
            ).wait()

            def gather_ring(i0, i1, rloc):
                return plsc.load_gather(ring.at[slot], [i0, i1, rloc])

            staged = extract(gather_ring, c * _CH, _CH, n_c, staged)
            return staged, n_s

        staged, _ = lax.fori_loop(0, n_ch, chunk_body, (0, 0))

        @pl.when(is0)
        def _():
            pltpu.sync_copy(tl, ring.at[0].at[:, :, pl.ds(0, 128)])
            n_c = compress(mpack, n_m, cpack, 23, _SEG // _CH)

            def gather_tail(i0, i1, rloc):
                return plsc.load_gather(ring.at[0], [i0, i1, rloc])

            final_drain(extract(gather_tail, _SEG, n_tail, n_c, staged))

        @pl.when(jnp.logical_not(is0))
        def _():
            final_drain(staged)

    return body(tableC, labels, tail2)


def kernel(labels, embedding_table):
    B, = labels.shape
    V, D = embedding_table.shape
    tableC = embedding_table.T.reshape(8, D // 8, V)
    tail_r0 = _TAIL0 * _SEG + 31 * _CH
    tail2 = jnp.pad(embedding_table[tail_r0:], ((0, 128 - (V - tail_r0)), (0, 0)))
    tail2 = tail2.T.reshape(8, D // 8, 128)
    out128 = _emb_lookup(tableC, labels.astype(jnp.int32), tail2, V, B)
    return out128[:B, :D]
